# baseline (speedup 1.0000x reference)
import functools

import jax
import jax.numpy as jnp
from jax import lax
from jax.experimental import pallas as pl
from jax.experimental.pallas import tpu as pltpu
from jax.experimental.pallas import tpu_sc as plsc

_B, _K, _R, _C = 128, 512, 196, 384
_NC, _NS = 2, 16
_NW = _NC * _NS
_RV = _R * _C // 128
_NR = 84
_CPS = _RV // _NR
_SPW = _B // _NW
_TPW = _SPW * _CPS
_NBUF = 2


def _sc_body(xid_hbm, gid_hbm, x_hbm, cen_hbm, off_hbm, out_hbm, loss_hbm,
             xidbuf, gidbuf, xbuf, offbuf, cenbuf, outbuf, lossv,
             xsem, fsem, csem, osem):
    wid = lax.axis_index("s") * _NC + lax.axis_index("c")

    pltpu.sync_copy(xid_hbm.at[wid], xidbuf)
    pltpu.sync_copy(gid_hbm.at[wid], gidbuf)

    def in_copies(t, s):
        return (
            pltpu.make_async_copy(
                x_hbm.at[xidbuf.at[t]], xbuf.at[s], xsem.at[s]),
            pltpu.make_async_copy(
                off_hbm.at[gidbuf.at[t]], offbuf.at[s], fsem.at[s]),
            pltpu.make_async_copy(
                cen_hbm.at[gidbuf.at[t]], cenbuf.at[s], csem.at[s]),
        )

    def start_in(t, s):
        for cp in in_copies(t, s):
            cp.start()

    def wait_in(t, s):
        for cp in in_copies(t, s):
            cp.wait()

    def out_copy(t, s):
        return pltpu.make_async_copy(
            outbuf.at[s], out_hbm.at[xidbuf.at[t]], osem.at[s])

    def add_chunk(s):
        def body(r, carry):
            for g in range(8):
                c = pl.ds(16 * g, 16)
                outbuf[s, r, c] = xbuf[s, r, c] + offbuf[s, r, c]
            return carry
        lax.fori_loop(0, _NR, body, 0)

    def sumsq(s, acc):
        def body(r, a):
            for g in range(8):
                v = cenbuf[s, r, pl.ds(16 * g, 16)]
                a = a + v * v
            return a
        return lax.fori_loop(0, _NR, body, acc)

    start_in(0, 0)
    start_in(1, 1)

    def step(g, acc):
        for b in range(_NBUF):
            t = _NBUF * g + b
            wait_in(t, b)

            @pl.when(t >= _NBUF)
            def _():
                out_copy(t - _NBUF, b).wait()

            add_chunk(b)
            out_copy(t, b).start()

            @pl.when(t + _NBUF < _TPW)
            def _():
                start_in(t + _NBUF, b)

            acc = sumsq(b, acc)
        return acc

    acc = lax.fori_loop(0, _TPW // _NBUF, step,
                        jnp.zeros((16,), jnp.float32))

    out_copy(_TPW - 2, 0).wait()
    out_copy(_TPW - 1, 1).wait()

    lossv[...] = acc
    pltpu.sync_copy(lossv, loss_hbm.at[wid])


_sc_kernel = functools.partial(
    pl.kernel,
    out_type=[
        jax.ShapeDtypeStruct((_B * _RV, 128), jnp.float32),
        jax.ShapeDtypeStruct((_NW, 16), jnp.float32),
    ],
    mesh=plsc.VectorSubcoreMesh(core_axis_name="c", subcore_axis_name="s"),
    scratch_types=[
        pltpu.VMEM((_TPW, _NR), jnp.int32),
        pltpu.VMEM((_TPW, _NR), jnp.int32),
        pltpu.VMEM((_NBUF, _NR, 128), jnp.float32),
        pltpu.VMEM((_NBUF, _NR, 128), jnp.float32),
        pltpu.VMEM((_NBUF, _NR, 128), jnp.float32),
        pltpu.VMEM((_NBUF, _NR, 128), jnp.float32),
        pltpu.VMEM((16,), jnp.float32),
        pltpu.SemaphoreType.DMA((_NBUF,)),
        pltpu.SemaphoreType.DMA((_NBUF,)),
        pltpu.SemaphoreType.DMA((_NBUF,)),
        pltpu.SemaphoreType.DMA((_NBUF,)),
    ],
)(_sc_body)


def kernel(x, identity, identity_centers, identity_offsets):
    idx = identity.astype(jnp.int32)
    chunk0 = (jnp.arange(_CPS, dtype=jnp.int32) * _NR)[None, :, None]
    row = jnp.arange(_NR, dtype=jnp.int32)[None, None, :]
    b_idx = jnp.arange(_B, dtype=jnp.int32).reshape(_NW, _SPW)
    xids = (b_idx[:, :, None, None] * _RV + chunk0 + row).reshape(
        _NW, _TPW, _NR)
    gids = (idx.reshape(_NW, _SPW)[:, :, None, None] * _RV + chunk0
            + row).reshape(_NW, _TPW, _NR)
    x_v = x.reshape(_B * _RV, 128)
    cen_v = identity_centers.reshape(_K * _RV, 128)
    off_v = identity_offsets.reshape(_K * _RV, 128)
    out, partial = _sc_kernel(xids, gids, x_v, cen_v, off_v)
    loss = jnp.sum(partial) * (1.0 / (_B * _R * _C))
    return out.reshape(_B, _R, _C), loss


# speedup vs baseline: 1.0018x; 1.0018x over previous
import functools

import jax
import jax.numpy as jnp
from jax import lax
from jax.experimental import pallas as pl
from jax.experimental.pallas import tpu as pltpu
from jax.experimental.pallas import tpu_sc as plsc

_B, _K, _R, _C = 128, 512, 196, 384
_NC, _NS = 2, 16
_NW = _NC * _NS
_RV = _R * _C // 128
_NR = 84
_CPS = _RV // _NR
_SPW = _B // _NW
_TPW = _SPW * _CPS
_NBUF = 2


def _sc_body(xid_hbm, gid_hbm, x_hbm, cen_hbm, off_hbm, out_hbm, loss_hbm,
             xidbuf, gidbuf, xbuf, offbuf, cenbuf, outbuf, lossv,
             xsem, fsem, csem, osem):
    wid = lax.axis_index("s") * _NC + lax.axis_index("c")

    pltpu.sync_copy(xid_hbm.at[wid], xidbuf)
    pltpu.sync_copy(gid_hbm.at[wid], gidbuf)

    def in_copies(t, s):
        return (
            pltpu.make_async_copy(
                x_hbm.at[xidbuf.at[t]], xbuf.at[s], xsem.at[s]),
            pltpu.make_async_copy(
                off_hbm.at[gidbuf.at[t]], offbuf.at[s], fsem.at[s]),
            pltpu.make_async_copy(
                cen_hbm.at[gidbuf.at[t]], cenbuf.at[s], csem.at[s]),
        )

    def start_in(t, s):
        for cp in in_copies(t, s):
            cp.start()

    def wait_in(t, s):
        for cp in in_copies(t, s):
            cp.wait()

    def out_copy(t, s):
        return pltpu.make_async_copy(
            outbuf.at[s], out_hbm.at[xidbuf.at[t]], osem.at[s])

    def compute(s, acc0):
        @plsc.parallel_loop(0, _NR, carry=acc0, unroll=4)
        def body(r, acc):
            for g in range(8):
                c = pl.ds(16 * g, 16)
                outbuf[s, r, c] = xbuf[s, r, c] + offbuf[s, r, c]
                v = cenbuf[s, r, c]
                acc = acc + v * v
            return acc
        return body

    start_in(0, 0)
    start_in(1, 1)

    def step(g, acc):
        for b in range(_NBUF):
            t = _NBUF * g + b
            wait_in(t, b)

            @pl.when(t >= _NBUF)
            def _():
                out_copy(t - _NBUF, b).wait()

            acc = compute(b, acc)
            out_copy(t, b).start()

            @pl.when(t + _NBUF < _TPW)
            def _():
                start_in(t + _NBUF, b)
        return acc

    acc = lax.fori_loop(0, _TPW // _NBUF, step,
                        jnp.zeros((16,), jnp.float32))

    out_copy(_TPW - 2, 0).wait()
    out_copy(_TPW - 1, 1).wait()

    lossv[...] = acc
    pltpu.sync_copy(lossv, loss_hbm.at[wid])


_sc_kernel = functools.partial(
    pl.kernel,
    out_type=[
        jax.ShapeDtypeStruct((_B * _RV, 128), jnp.float32),
        jax.ShapeDtypeStruct((_NW, 16), jnp.float32),
    ],
    mesh=plsc.VectorSubcoreMesh(core_axis_name="c", subcore_axis_name="s"),
    scratch_types=[
        pltpu.VMEM((_TPW, _NR), jnp.int32),
        pltpu.VMEM((_TPW, _NR), jnp.int32),
        pltpu.VMEM((_NBUF, _NR, 128), jnp.float32),
        pltpu.VMEM((_NBUF, _NR, 128), jnp.float32),
        pltpu.VMEM((_NBUF, _NR, 128), jnp.float32),
        pltpu.VMEM((_NBUF, _NR, 128), jnp.float32),
        pltpu.VMEM((16,), jnp.float32),
        pltpu.SemaphoreType.DMA((_NBUF,)),
        pltpu.SemaphoreType.DMA((_NBUF,)),
        pltpu.SemaphoreType.DMA((_NBUF,)),
        pltpu.SemaphoreType.DMA((_NBUF,)),
    ],
)(_sc_body)


def kernel(x, identity, identity_centers, identity_offsets):
    idx = identity.astype(jnp.int32)
    chunk0 = (jnp.arange(_CPS, dtype=jnp.int32) * _NR)[None, :, None]
    row = jnp.arange(_NR, dtype=jnp.int32)[None, None, :]
    b_idx = jnp.arange(_B, dtype=jnp.int32).reshape(_NW, _SPW)
    xids = (b_idx[:, :, None, None] * _RV + chunk0 + row).reshape(
        _NW, _TPW, _NR)
    gids = (idx.reshape(_NW, _SPW)[:, :, None, None] * _RV + chunk0
            + row).reshape(_NW, _TPW, _NR)
    x_v = x.reshape(_B * _RV, 128)
    cen_v = identity_centers.reshape(_K * _RV, 128)
    off_v = identity_offsets.reshape(_K * _RV, 128)
    out, partial = _sc_kernel(xids, gids, x_v, cen_v, off_v)
    loss = jnp.sum(partial) * (1.0 / (_B * _R * _C))
    return out.reshape(_B, _R, _C), loss


# speedup vs baseline: 1.9701x; 1.9666x over previous
import functools

import jax
import jax.numpy as jnp
from jax import lax
from jax.experimental import pallas as pl
from jax.experimental.pallas import tpu as pltpu
from jax.experimental.pallas import tpu_sc as plsc

_B, _K, _R, _C = 128, 512, 196, 384
_NC, _NS = 2, 16
_NW = _NC * _NS
_SPW = _B // _NW
_RF = 32
_NFC = 6
_RL = _R - _NFC * _RF
_TPF = _SPW * _NFC
_G = _C // 16
_NBUF = 2


def _sc_body(id_hbm, x_hbm, cen_hbm, off_hbm, out_hbm, loss_hbm,
             idv, xbuf, offbuf, cenbuf, outbuf, lossv,
             xsem, fsem, csem, osem):
    wid = lax.axis_index("s") * _NC + lax.axis_index("c")
    b0 = wid * _SPW

    pltpu.sync_copy(id_hbm, idv.at[pl.ds(0, _B)])

    def _id(b):
        return idv[pl.ds(b, 16)][0]

    def full_copies(t, s):
        bl = t // _NFC
        r0 = pl.multiple_of((t % _NFC) * _RF, _RF)
        b = b0 + bl
        k = _id(b)
        return (
            pltpu.make_async_copy(
                x_hbm.at[b, pl.ds(r0, _RF)], xbuf.at[s], xsem.at[s]),
            pltpu.make_async_copy(
                off_hbm.at[k, pl.ds(r0, _RF)], offbuf.at[s], fsem.at[s]),
            pltpu.make_async_copy(
                cen_hbm.at[k, pl.ds(r0, _RF)], cenbuf.at[s], csem.at[s]),
        )

    def out_full(t, s):
        bl = t // _NFC
        r0 = pl.multiple_of((t % _NFC) * _RF, _RF)
        return pltpu.make_async_copy(
            outbuf.at[s], out_hbm.at[b0 + bl, pl.ds(r0, _RF)], osem.at[s])

    def tail_copies(u, s):
        b = b0 + u
        k = _id(b)
        return (
            pltpu.make_async_copy(
                x_hbm.at[b, pl.ds(_NFC * _RF, _RL)],
                xbuf.at[s, pl.ds(0, _RL)], xsem.at[s]),
            pltpu.make_async_copy(
                off_hbm.at[k, pl.ds(_NFC * _RF, _RL)],
                offbuf.at[s, pl.ds(0, _RL)], fsem.at[s]),
            pltpu.make_async_copy(
                cen_hbm.at[k, pl.ds(_NFC * _RF, _RL)],
                cenbuf.at[s, pl.ds(0, _RL)], csem.at[s]),
        )

    def out_tail(u, s):
        return pltpu.make_async_copy(
            outbuf.at[s, pl.ds(0, _RL)],
            out_hbm.at[b0 + u, pl.ds(_NFC * _RF, _RL)], osem.at[s])

    def compute(s, nrows, acc0):
        @plsc.parallel_loop(0, nrows, carry=acc0, unroll=2)
        def body(r, acc):
            for g in range(_G):
                c = pl.ds(16 * g, 16)
                outbuf[s, r, c] = xbuf[s, r, c] + offbuf[s, r, c]
                v = cenbuf[s, r, c]
                acc = acc + v * v
            return acc
        return body

    for cp in full_copies(0, 0) + full_copies(1, 1):
        cp.start()

    def step(g, acc):
        for b in range(_NBUF):
            t = _NBUF * g + b
            for cp in full_copies(t, b):
                cp.wait()

            @pl.when(t >= _NBUF)
            def _():
                out_full(t - _NBUF, b).wait()

            acc = compute(b, _RF, acc)
            out_full(t, b).start()

            @pl.when(t + _NBUF < _TPF)
            def _():
                for cp in full_copies(t + _NBUF, b):
                    cp.start()
        return acc

    acc = lax.fori_loop(0, _TPF // _NBUF, step,
                        jnp.zeros((16,), jnp.float32))

    for cp in tail_copies(0, 0) + tail_copies(1, 1):
        cp.start()
    for u in range(_SPW):
        s = u % _NBUF
        for cp in tail_copies(u, s):
            cp.wait()
        if u < _NBUF:
            out_full(_TPF - _NBUF + u, s).wait()
        else:
            out_tail(u - _NBUF, s).wait()
        acc = compute(s, _RL, acc)
        out_tail(u, s).start()
        if u + _NBUF < _SPW:
            for cp in tail_copies(u + _NBUF, s):
                cp.start()

    out_tail(_SPW - 2, (_SPW - 2) % _NBUF).wait()
    out_tail(_SPW - 1, (_SPW - 1) % _NBUF).wait()

    lossv[...] = acc
    pltpu.sync_copy(lossv, loss_hbm.at[wid])


_sc_kernel = functools.partial(
    pl.kernel,
    out_type=[
        jax.ShapeDtypeStruct((_B, _R, _C), jnp.float32),
        jax.ShapeDtypeStruct((_NW, 16), jnp.float32),
    ],
    mesh=plsc.VectorSubcoreMesh(core_axis_name="c", subcore_axis_name="s"),
    scratch_types=[
        pltpu.VMEM((_B + 16,), jnp.int32),
        pltpu.VMEM((_NBUF, _RF, _C), jnp.float32),
        pltpu.VMEM((_NBUF, _RF, _C), jnp.float32),
        pltpu.VMEM((_NBUF, _RF, _C), jnp.float32),
        pltpu.VMEM((_NBUF, _RF, _C), jnp.float32),
        pltpu.VMEM((16,), jnp.float32),
        pltpu.SemaphoreType.DMA((_NBUF,)),
        pltpu.SemaphoreType.DMA((_NBUF,)),
        pltpu.SemaphoreType.DMA((_NBUF,)),
        pltpu.SemaphoreType.DMA((_NBUF,)),
    ],
)(_sc_body)


def kernel(x, identity, identity_centers, identity_offsets):
    idx = identity.astype(jnp.int32)
    out, partial = _sc_kernel(idx, x, identity_centers, identity_offsets)
    loss = jnp.sum(partial) * (1.0 / (_B * _R * _C))
    return out, loss
